# baseline (speedup 1.0000x reference)
import functools

import numpy as np
import jax
import jax.numpy as jnp
from jax import lax
from jax.experimental import pallas as pl
from jax.experimental.pallas import tpu as pltpu
from jax.experimental.pallas import tpu_sc as plsc

MAXL = 8192
BATCH = 2
D = 768
FLAT = MAXL * BATCH
NC, NS, LANES = 2, 16, 16
NW = NC * NS
PER_W = FLAT // NW
CHUNK = 64
NCHUNK = PER_W // CHUNK
POS_PER_W = PER_W // 2
POS_CHUNK = CHUNK // 2
NGRP = D // LANES


def _make_pos() -> np.ndarray:
    pos = np.empty((MAXL, D), dtype=np.float32)
    theta_even = np.arange(MAXL, dtype=np.float32)[:, None] / (
        10000.0 ** (2.0 * np.arange((D + 1) // 2, dtype=np.float32)[None, :] / D))
    theta_odd = np.arange(MAXL, dtype=np.float32)[:, None] / (
        10000.0 ** (2.0 * np.arange(D // 2, dtype=np.float32)[None, :] / D))
    pos[:, 0::2] = np.sin(theta_even)
    pos[:, 1::2] = np.cos(theta_odd)
    return pos


_POS = _make_pos()


@functools.partial(
    pl.kernel,
    out_type=jax.ShapeDtypeStruct((FLAT, D), jnp.float32),
    mesh=plsc.VectorSubcoreMesh(core_axis_name="c", subcore_axis_name="s"),
    scratch_types=[
        pltpu.VMEM((NCHUNK, CHUNK), jnp.int32),
        pltpu.VMEM((POS_CHUNK, D), jnp.float32),
        pltpu.VMEM((CHUNK, D), jnp.float32),
        pltpu.SemaphoreType.DMA,
    ],
)
def _emb_kernel(x_hbm, pos_hbm, table_hbm, out_hbm, idx_v, pos_v, rows_v, gsem):
    wid = lax.axis_index("s") * NC + lax.axis_index("c")
    base = wid * PER_W
    pbase = wid * POS_PER_W

    pltpu.sync_copy(x_hbm.at[wid], idx_v)

    for j in range(NCHUNK):
        gcopy = pltpu.async_copy(table_hbm.at[idx_v.at[j]], rows_v, gsem)
        pltpu.sync_copy(
            pos_hbm.at[pl.ds(pbase + j * POS_CHUNK, POS_CHUNK)], pos_v)
        gcopy.wait()

        def row_body(p, carry):
            r0 = 2 * p
            r1 = 2 * p + 1
            for g in range(NGRP):
                sl = pl.ds(g * LANES, LANES)
                pv = pos_v[p, sl]
                a = rows_v[r0, sl]
                b = rows_v[r1, sl]
                rows_v[r0, sl] = a + a + pv
                rows_v[r1, sl] = b + b + pv
            return carry

        lax.fori_loop(0, POS_CHUNK, row_body, 0)

        pltpu.sync_copy(rows_v, out_hbm.at[pl.ds(base + j * CHUNK, CHUNK)])


def kernel(x, table):
    xi = x.astype(jnp.int32).reshape(NW, NCHUNK, CHUNK)
    pos = jnp.asarray(_POS)
    out = _emb_kernel(xi, pos, table)
    return out.reshape(MAXL, BATCH, D)


# speedup vs baseline: 1.3152x; 1.3152x over previous
import functools

import numpy as np
import jax
import jax.numpy as jnp
from jax import lax
from jax.experimental import pallas as pl
from jax.experimental.pallas import tpu as pltpu
from jax.experimental.pallas import tpu_sc as plsc

MAXL = 8192
BATCH = 2
D = 768
FLAT = MAXL * BATCH
NC, NS, LANES = 2, 16, 16
NW = NC * NS
PER_W = FLAT // NW
CHUNK = 32
NCHUNK = PER_W // CHUNK
POS_PER_W = PER_W // 2
POS_CHUNK = CHUNK // 2
NGRP = D // LANES
SLOTS = 3


def _make_pos() -> np.ndarray:
    pos = np.empty((MAXL, D), dtype=np.float32)
    theta_even = np.arange(MAXL, dtype=np.float32)[:, None] / (
        10000.0 ** (2.0 * np.arange((D + 1) // 2, dtype=np.float32)[None, :] / D))
    theta_odd = np.arange(MAXL, dtype=np.float32)[:, None] / (
        10000.0 ** (2.0 * np.arange(D // 2, dtype=np.float32)[None, :] / D))
    pos[:, 0::2] = np.sin(theta_even)
    pos[:, 1::2] = np.cos(theta_odd)
    return pos


_POS = _make_pos()


@functools.partial(
    pl.kernel,
    out_type=jax.ShapeDtypeStruct((FLAT, D), jnp.float32),
    mesh=plsc.VectorSubcoreMesh(core_axis_name="c", subcore_axis_name="s"),
    scratch_types=(
        [pltpu.VMEM((NCHUNK, CHUNK), jnp.int32)]
        + [pltpu.VMEM((CHUNK, D), jnp.float32) for _ in range(SLOTS)]
        + [pltpu.VMEM((POS_CHUNK, D), jnp.float32) for _ in range(SLOTS)]
        + [pltpu.SemaphoreType.DMA for _ in range(2 * SLOTS)]
    ),
)
def _emb_kernel(x_hbm, pos_hbm, table_hbm, out_hbm, idx_v,
                rows0, rows1, rows2, pos0, pos1, pos2,
                gsem0, gsem1, gsem2, osem0, osem1, osem2):
    rows = (rows0, rows1, rows2)
    posb = (pos0, pos1, pos2)
    gsem = (gsem0, gsem1, gsem2)
    osem = (osem0, osem1, osem2)

    wid = lax.axis_index("s") * NC + lax.axis_index("c")
    base = wid * PER_W
    pbase = wid * POS_PER_W

    pltpu.sync_copy(x_hbm.at[wid], idx_v)

    def start(j):
        s = j % SLOTS
        g = pltpu.async_copy(table_hbm.at[idx_v.at[j]], rows[s], gsem[s])
        p = pltpu.async_copy(
            pos_hbm.at[pl.ds(pbase + j * POS_CHUNK, POS_CHUNK)],
            posb[s], gsem[s])
        return (g, p)

    descs = [None] * NCHUNK
    odescs = [None] * NCHUNK
    descs[0] = start(0)
    descs[1] = start(1)

    for j in range(NCHUNK):
        s = j % SLOTS
        if j + 1 >= 2 and j + 1 < NCHUNK:
            if j - 2 >= 0:
                odescs[j - 2].wait()
            descs[j + 1] = start(j + 1)

        g, p = descs[j]
        g.wait()
        p.wait()

        rs = rows[s]
        ps = posb[s]

        def row_body(prow, carry):
            r0 = 2 * prow
            r1 = r0 + 1

            @plsc.parallel_loop(0, NGRP, unroll=4)
            def _(grp):
                sl = pl.ds(grp * LANES, LANES)
                pv = ps[prow, sl]
                a = rs[r0, sl]
                b = rs[r1, sl]
                rs[r0, sl] = a + a + pv
                rs[r1, sl] = b + b + pv

            return carry

        lax.fori_loop(0, POS_CHUNK, row_body, 0)

        odescs[j] = pltpu.async_copy(
            rs, out_hbm.at[pl.ds(base + j * CHUNK, CHUNK)], osem[s])

    for j in range(NCHUNK - SLOTS, NCHUNK):
        odescs[j].wait()


def kernel(x, table):
    xi = x.astype(jnp.int32).reshape(NW, NCHUNK, CHUNK)
    pos = jnp.asarray(_POS)
    out = _emb_kernel(xi, pos, table)
    return out.reshape(MAXL, BATCH, D)


# speedup vs baseline: 3.4092x; 2.5921x over previous
import functools

import numpy as np
import jax
import jax.numpy as jnp
from jax import lax
from jax.experimental import pallas as pl
from jax.experimental.pallas import tpu as pltpu
from jax.experimental.pallas import tpu_sc as plsc

MAXL = 8192
BATCH = 2
D = 768
NC, NS, LANES = 2, 16, 16
NW = NC * NS
L_PER_W = MAXL // NW
PC = 16
NCHUNK = L_PER_W // PC
NGRP = D // LANES
SLOTS = 3


def _make_pos() -> np.ndarray:
    pos = np.empty((MAXL, D), dtype=np.float32)
    theta_even = np.arange(MAXL, dtype=np.float32)[:, None] / (
        10000.0 ** (2.0 * np.arange((D + 1) // 2, dtype=np.float32)[None, :] / D))
    theta_odd = np.arange(MAXL, dtype=np.float32)[:, None] / (
        10000.0 ** (2.0 * np.arange(D // 2, dtype=np.float32)[None, :] / D))
    pos[:, 0::2] = np.sin(theta_even)
    pos[:, 1::2] = np.cos(theta_odd)
    return pos


_POS = _make_pos()


@functools.partial(
    pl.kernel,
    out_type=jax.ShapeDtypeStruct((MAXL, BATCH, D), jnp.float32),
    mesh=plsc.VectorSubcoreMesh(core_axis_name="c", subcore_axis_name="s"),
    scratch_types=(
        [pltpu.VMEM((NCHUNK, BATCH, PC), jnp.int32)]
        + [pltpu.VMEM((BATCH, PC, D), jnp.float32) for _ in range(SLOTS)]
        + [pltpu.VMEM((PC, D), jnp.float32) for _ in range(SLOTS)]
        + [pltpu.SemaphoreType.DMA for _ in range(2 * SLOTS)]
    ),
)
def _emb_kernel(x_hbm, pos_hbm, table_hbm, out_hbm, idx_v,
                rows0, rows1, rows2, pos0, pos1, pos2,
                gsem0, gsem1, gsem2, osem0, osem1, osem2):
    rows = (rows0, rows1, rows2)
    posb = (pos0, pos1, pos2)
    gsem = (gsem0, gsem1, gsem2)
    osem = (osem0, osem1, osem2)

    wid = lax.axis_index("s") * NC + lax.axis_index("c")
    lbase = wid * L_PER_W

    pltpu.sync_copy(x_hbm.at[wid], idx_v)

    def start(j):
        s = j % SLOTS
        g0 = pltpu.async_copy(table_hbm.at[idx_v.at[j, 0]], rows[s].at[0],
                              gsem[s])
        g1 = pltpu.async_copy(table_hbm.at[idx_v.at[j, 1]], rows[s].at[1],
                              gsem[s])
        p = pltpu.async_copy(pos_hbm.at[pl.ds(lbase + j * PC, PC)],
                             posb[s], gsem[s])
        return (g0, g1, p)

    descs = [None] * NCHUNK
    odescs = [None] * NCHUNK
    descs[0] = start(0)
    descs[1] = start(1)

    for j in range(NCHUNK):
        s = j % SLOTS
        if j + 1 >= 2 and j + 1 < NCHUNK:
            if j - 2 >= 0:
                for od in odescs[j - 2]:
                    od.wait()
            descs[j + 1] = start(j + 1)

        for dsc in descs[j]:
            dsc.wait()

        rs = rows[s]
        ps = posb[s]

        def row_body(t, carry):
            @plsc.parallel_loop(0, NGRP, unroll=4)
            def _(grp):
                sl = pl.ds(grp * LANES, LANES)
                pv = ps[t, sl]
                a = rs[0, t, sl]
                b = rs[1, t, sl]
                rs[0, t, sl] = a + a + pv
                rs[1, t, sl] = b + b + pv

            return carry

        lax.fori_loop(0, PC, row_body, 0)

        l0 = lbase + j * PC
        odescs[j] = (
            pltpu.async_copy(rs.at[0], out_hbm.at[pl.ds(l0, PC), 0], osem[s]),
            pltpu.async_copy(rs.at[1], out_hbm.at[pl.ds(l0, PC), 1], osem[s]),
        )

    for j in range(NCHUNK - SLOTS, NCHUNK):
        for od in odescs[j]:
            od.wait()


def kernel(x, table):
    xi = (x.astype(jnp.int32)
          .reshape(NW, NCHUNK, PC, BATCH)
          .transpose(0, 1, 3, 2))
    pos = jnp.asarray(_POS)
    return _emb_kernel(xi, pos, table)


# speedup vs baseline: 3.6216x; 1.0623x over previous
import functools

import numpy as np
import jax
import jax.numpy as jnp
from jax import lax
from jax.experimental import pallas as pl
from jax.experimental.pallas import tpu as pltpu
from jax.experimental.pallas import tpu_sc as plsc

MAXL = 8192
BATCH = 2
D = 768
NC, NS, LANES = 2, 16, 16
NW = NC * NS
L_PER_W = MAXL // NW
PC = 16
CHUNK = BATCH * PC
NCHUNK = L_PER_W // PC
NGRP = D // LANES
SLOTS = 4


def _make_pos_factors():
    d = np.arange(D)
    w = 10000.0 ** (-2.0 * (d // 2) / D)
    l0 = (np.arange(NW * NCHUNK) * PC)[:, None]
    even = (d % 2 == 0)
    u = np.where(even, np.sin(l0 * w), np.cos(l0 * w))
    v = np.where(even, np.cos(l0 * w), -np.sin(l0 * w))
    t = np.arange(PC)[:, None]
    c = np.cos(t * w)
    s = np.sin(t * w)
    uv = np.stack([u, v], axis=1).reshape(NW, NCHUNK, 2, D).astype(np.float32)
    cs = np.stack([c, s], axis=0).astype(np.float32)
    return uv, cs


_UV, _CS = _make_pos_factors()


@functools.partial(
    pl.kernel,
    out_type=jax.ShapeDtypeStruct((MAXL, BATCH, D), jnp.float32),
    mesh=plsc.VectorSubcoreMesh(core_axis_name="c", subcore_axis_name="s"),
    scratch_types=(
        [pltpu.VMEM((NCHUNK, CHUNK), jnp.int32),
         pltpu.VMEM((2, PC, D), jnp.float32)]
        + [pltpu.VMEM((CHUNK, D), jnp.float32) for _ in range(SLOTS)]
        + [pltpu.VMEM((2, D), jnp.float32) for _ in range(SLOTS)]
        + [pltpu.SemaphoreType.DMA for _ in range(2 * SLOTS)]
    ),
)
def _emb_kernel(x_hbm, uv_hbm, cs_hbm, table_hbm, out_hbm, idx_v, cs_v,
                rows0, rows1, rows2, rows3, uv0, uv1, uv2, uv3,
                gsem0, gsem1, gsem2, gsem3, osem0, osem1, osem2, osem3):
    rows = (rows0, rows1, rows2, rows3)
    uvb = (uv0, uv1, uv2, uv3)
    gsem = (gsem0, gsem1, gsem2, gsem3)
    osem = (osem0, osem1, osem2, osem3)

    wid = lax.axis_index("s") * NC + lax.axis_index("c")
    lbase = wid * L_PER_W

    pltpu.sync_copy(x_hbm.at[wid], idx_v)
    pltpu.sync_copy(cs_hbm, cs_v)

    def start(j):
        s = j % SLOTS
        g = pltpu.async_copy(table_hbm.at[idx_v.at[j]], rows[s], gsem[s])
        p = pltpu.async_copy(uv_hbm.at[wid, j], uvb[s], gsem[s])
        return (g, p)

    descs = [None] * NCHUNK
    odescs = [None] * NCHUNK
    descs[0] = start(0)
    descs[1] = start(1)

    for j in range(NCHUNK):
        s = j % SLOTS
        nxt = j + 2
        if nxt >= 2 and nxt < NCHUNK:
            if nxt - SLOTS >= 0:
                for od in odescs[nxt - SLOTS]:
                    od.wait()
            descs[nxt] = start(nxt)

        for dsc in descs[j]:
            dsc.wait()

        rs = rows[s]
        uvs = uvb[s]

        def grp_body(grp, carry):
            sl = pl.ds(grp * LANES, LANES)
            u = uvs[0, sl]
            v = uvs[1, sl]

            @plsc.parallel_loop(0, PC, unroll=4)
            def _(t):
                pv = u * cs_v[0, t, sl] + v * cs_v[1, t, sl]
                a = rs[t, sl]
                b = rs[PC + t, sl]
                rs[t, sl] = a + a + pv
                rs[PC + t, sl] = b + b + pv

            return carry

        lax.fori_loop(0, NGRP, grp_body, 0)

        l0 = lbase + j * PC
        odescs[j] = (
            pltpu.async_copy(rs.at[pl.ds(0, PC)],
                             out_hbm.at[pl.ds(l0, PC), 0], osem[s]),
            pltpu.async_copy(rs.at[pl.ds(PC, PC)],
                             out_hbm.at[pl.ds(l0, PC), 1], osem[s]),
        )

    for j in range(NCHUNK - SLOTS, NCHUNK):
        if j >= 0:
            for od in odescs[j]:
                od.wait()


def kernel(x, table):
    xi = (x.astype(jnp.int32)
          .reshape(NW, NCHUNK, PC, BATCH)
          .transpose(0, 1, 3, 2)
          .reshape(NW, NCHUNK, CHUNK))
    return _emb_kernel(xi, jnp.asarray(_UV), jnp.asarray(_CS), table)


# speedup vs baseline: 3.7148x; 1.0257x over previous
import functools

import numpy as np
import jax
import jax.numpy as jnp
from jax import lax
from jax.experimental import pallas as pl
from jax.experimental.pallas import tpu as pltpu
from jax.experimental.pallas import tpu_sc as plsc

MAXL = 8192
BATCH = 2
D = 768
NC, NS, LANES = 2, 16, 16
NW = NC * NS
L_PER_W = MAXL // NW
PC = 16
CHUNK = BATCH * PC
NCHUNK = L_PER_W // PC
NGRP = D // LANES
SLOTS = 3


def _make_pos_factors():
    d = np.arange(D)
    w = 10000.0 ** (-2.0 * (d // 2) / D)
    l0 = (np.arange(NW * NCHUNK) * PC)[:, None]
    even = (d % 2 == 0)
    u = np.where(even, np.sin(l0 * w), np.cos(l0 * w))
    v = np.where(even, np.cos(l0 * w), -np.sin(l0 * w))
    t = np.arange(PC)[:, None]
    c = np.cos(t * w)
    s = np.sin(t * w)
    uv = np.stack([u, v], axis=1).reshape(NW, NCHUNK, 2, D).astype(np.float32)
    cs = np.stack([c, s], axis=0).astype(np.float32)
    return uv, cs


_UV, _CS = _make_pos_factors()


@functools.partial(
    pl.kernel,
    out_type=jax.ShapeDtypeStruct((MAXL, BATCH, D), jnp.float32),
    mesh=plsc.VectorSubcoreMesh(core_axis_name="c", subcore_axis_name="s"),
    scratch_types=(
        [pltpu.VMEM((NCHUNK, CHUNK), jnp.int32),
         pltpu.VMEM((2, PC, D), jnp.float32)]
        + [pltpu.VMEM((CHUNK, D), jnp.float32) for _ in range(SLOTS)]
        + [pltpu.VMEM((2, D), jnp.float32) for _ in range(SLOTS)]
        + [pltpu.SemaphoreType.DMA for _ in range(2 * SLOTS)]
    ),
)
def _emb_kernel(x_hbm, uv_hbm, cs_hbm, table_hbm, out_hbm, idx_v, cs_v,
                rows0, rows1, rows2, uv0, uv1, uv2,
                gsem0, gsem1, gsem2, osem0, osem1, osem2):
    rows = (rows0, rows1, rows2)
    uvb = (uv0, uv1, uv2)
    gsem = (gsem0, gsem1, gsem2)
    osem = (osem0, osem1, osem2)

    wid = lax.axis_index("s") * NC + lax.axis_index("c")
    lbase = wid * L_PER_W

    pltpu.sync_copy(x_hbm.at[wid], idx_v)
    pltpu.sync_copy(cs_hbm, cs_v)

    def start(j):
        s = j % SLOTS
        g = pltpu.async_copy(table_hbm.at[idx_v.at[j]], rows[s], gsem[s])
        p = pltpu.async_copy(uv_hbm.at[wid, j], uvb[s], gsem[s])
        return (g, p)

    descs = [None] * NCHUNK
    odescs = [None] * NCHUNK
    descs[0] = start(0)
    descs[1] = start(1)

    for j in range(NCHUNK):
        s = j % SLOTS
        nxt = j + 1
        if nxt >= 2 and nxt < NCHUNK:
            if nxt - SLOTS >= 0:
                for od in odescs[nxt - SLOTS]:
                    od.wait()
            descs[nxt] = start(nxt)

        for dsc in descs[j]:
            dsc.wait()

        rs = rows[s]
        uvs = uvb[s]

        def grp_body(grp, carry):
            sl = pl.ds(grp * LANES, LANES)
            u = uvs[0, sl]
            v = uvs[1, sl]

            @plsc.parallel_loop(0, PC, unroll=4)
            def _(t):
                pv = u * cs_v[0, t, sl] + v * cs_v[1, t, sl]
                a = rs[t, sl]
                b = rs[PC + t, sl]
                rs[t, sl] = a + a + pv
                rs[PC + t, sl] = b + b + pv

            return carry

        lax.fori_loop(0, NGRP, grp_body, 0)

        l0 = lbase + j * PC
        odescs[j] = (
            pltpu.async_copy(rs.at[pl.ds(0, PC)],
                             out_hbm.at[pl.ds(l0, PC), 0], osem[s]),
            pltpu.async_copy(rs.at[pl.ds(PC, PC)],
                             out_hbm.at[pl.ds(l0, PC), 1], osem[s]),
        )

    for j in range(NCHUNK - SLOTS, NCHUNK):
        if j >= 0:
            for od in odescs[j]:
                od.wait()


def kernel(x, table):
    xi = (x.astype(jnp.int32)
          .reshape(NW, NCHUNK, PC, BATCH)
          .transpose(0, 1, 3, 2)
          .reshape(NW, NCHUNK, CHUNK))
    return _emb_kernel(xi, jnp.asarray(_UV), jnp.asarray(_CS), table)
